# baseline (speedup 1.0000x reference)
import functools

import jax
import jax.numpy as jnp
from jax import lax
from jax.experimental import pallas as pl
from jax.experimental.pallas import tpu as pltpu
from jax.experimental.pallas import tpu_sc as plsc

NUM_CLASSES = 1000000
EMBED_DIM = 64
BATCH = 16384

NC = 2
NS = 16
NW = NC * NS

B_PER_W = BATCH // NW
LANES = 128
CHUNK_K = 4
CW = CHUNK_K * LANES
NBLK = (NUM_CLASSES + LANES - 1) // LANES
MAX_K0 = NBLK - CHUNK_K


@functools.partial(
    pl.kernel,
    mesh=plsc.VectorSubcoreMesh(core_axis_name="c", subcore_axis_name="s"),
    out_type=jax.ShapeDtypeStruct((BATCH, EMBED_DIM), jnp.float32),
    scratch_types=[
        pltpu.VMEM((B_PER_W,), jnp.int32),
        pltpu.VMEM((B_PER_W,), jnp.int32),
        pltpu.VMEM((2, EMBED_DIM, CW), jnp.float32),
        pltpu.VMEM((16, EMBED_DIM), jnp.float32),
        pltpu.VMEM_SHARED((NS, EMBED_DIM, CW), jnp.float32),
        pltpu.SemaphoreType.DMA,
        pltpu.SemaphoreType.DMA,
        pltpu.SemaphoreType.DMA,
        pltpu.SemaphoreType.DMA,
    ],
    compiler_params=pltpu.CompilerParams(needs_layout_passes=False),
)
def _gather_scan(table_hbm, idx_hbm, ord_hbm, out_hbm,
                 idx_v, ord_v, buf_v, ring_v, sh_v, sem_a, sem_b, sem_s, out_sem):
    wid = lax.axis_index("s") * NC + lax.axis_index("c")
    base = wid * B_PER_W
    pltpu.sync_copy(idx_hbm.at[pl.ds(base, B_PER_W)], idx_v)
    pltpu.sync_copy(ord_hbm.at[pl.ds(base, B_PER_W)], ord_v)

    lane_ids = jnp.arange(16, dtype=jnp.int32)
    row_ids = [lane_ids + 16 * t for t in range(EMBED_DIM // 16)]
    zeros16 = jnp.zeros((16,), jnp.int32)

    def at(ref, i):
        return plsc.load_gather(ref, [jnp.full((16,), i, jnp.int32)])[0]

    kf = at(idx_v, 0) >> 7
    klast = at(idx_v, B_PER_W - 1) >> 7
    n_c = ((klast - kf) >> 2) + 1
    kf128 = kf * LANES

    def kstart(c):
        return pl.multiple_of(jnp.minimum(kf + CHUNK_K * c, MAX_K0) * LANES, LANES)

    def fire(c, p, sem):
        pltpu.async_copy(
            table_hbm.at[:, pl.ds(kstart(c), CW)], buf_v.at[p], sem
        )

    def drain(p, sem):
        pltpu.make_async_copy(
            table_hbm.at[:, pl.ds(0, CW)], buf_v.at[p], sem
        ).wait()

    def extract_window(c, p, ptr):
        ks0 = kstart(c)
        be = kf128 + (c + 1) * CW

        def cond(i):
            lab = at(idx_v, jnp.minimum(i, B_PER_W - 1))
            return (i < B_PER_W) & (lab < be)

        def lab_body(i):
            lab = at(idx_v, i)
            bo = at(ord_v, i)
            col = jnp.full((16,), lab - ks0, jnp.int32)
            slot = lax.rem(i, 16)
            sl16 = jnp.full((16,), slot, jnp.int32)
            for t in range(EMBED_DIM // 16):
                vals = plsc.load_gather(buf_v.at[p], [row_ids[t], col])
                plsc.store_scatter(ring_v, [sl16, row_ids[t]], vals)

            @pl.when(i >= 16)
            def _():
                pltpu.make_async_copy(
                    out_hbm.at[pl.ds(base, 1)], ring_v.at[pl.ds(0, 1)], out_sem
                ).wait()

            pltpu.async_copy(
                ring_v.at[pl.ds(slot, 1)], out_hbm.at[pl.ds(bo, 1)], out_sem
            )
            return i + 1

        return lax.while_loop(cond, lab_body, ptr)

    fire(0, 0, sem_a)

    sid = lax.axis_index("s")

    def body(c, ptr):
        @pl.when(c >= 1)
        def _():
            pltpu.make_async_copy(
                table_hbm.at[:, pl.ds(0, CW)], sh_v.at[sid], sem_s
            ).wait()

        @pl.when(c + 1 < n_c)
        def _():
            pltpu.async_copy(
                table_hbm.at[:, pl.ds(kstart(c + 1), CW)], sh_v.at[sid], sem_s
            )

        def even(ptr):
            @pl.when(c + 1 < n_c)
            def _():
                fire(c + 1, 1, sem_b)

            drain(0, sem_a)
            return extract_window(c, 0, ptr)

        def odd(ptr):
            @pl.when(c + 1 < n_c)
            def _():
                fire(c + 1, 0, sem_a)

            drain(1, sem_b)
            return extract_window(c, 1, ptr)

        return lax.cond(lax.rem(c, 2) == 0, even, odd, ptr)

    lax.fori_loop(0, n_c, body, 0)

    for _ in range(16):
        pltpu.make_async_copy(
            out_hbm.at[pl.ds(base, 1)], ring_v.at[pl.ds(0, 1)], out_sem
        ).wait()


def kernel(labels, embedding_table):
    iota = jnp.arange(BATCH, dtype=jnp.int32)
    slab, order = lax.sort((labels.astype(jnp.int32), iota), num_keys=1)
    return _gather_scan(embedding_table.T, slab, order)


# speedup vs baseline: 1.3886x; 1.3886x over previous
import functools

import jax
import jax.numpy as jnp
from jax import lax
from jax.experimental import pallas as pl
from jax.experimental.pallas import tpu as pltpu
from jax.experimental.pallas import tpu_sc as plsc

NUM_CLASSES = 1000000
EMBED_DIM = 64
BATCH = 16384

NC = 2
NS = 16
NW = NC * NS

B_PER_W = BATCH // NW
LANES = 128
CHUNK_K = 6
CW = CHUNK_K * LANES
NBLK = (NUM_CLASSES + LANES - 1) // LANES
MAX_K0 = NBLK - CHUNK_K


@functools.partial(
    pl.kernel,
    mesh=plsc.VectorSubcoreMesh(core_axis_name="c", subcore_axis_name="s"),
    out_type=jax.ShapeDtypeStruct((BATCH, EMBED_DIM), jnp.float32),
    scratch_types=[
        pltpu.VMEM((B_PER_W,), jnp.int32),
        pltpu.VMEM((B_PER_W,), jnp.int32),
        pltpu.VMEM((2, EMBED_DIM, CW), jnp.float32),
        pltpu.VMEM((16, EMBED_DIM), jnp.float32),
        pltpu.SemaphoreType.DMA,
        pltpu.SemaphoreType.DMA,
        pltpu.SemaphoreType.DMA,
    ],
    compiler_params=pltpu.CompilerParams(needs_layout_passes=False),
)
def _gather_scan(table_hbm, idx_hbm, ord_hbm, out_hbm,
                 idx_v, ord_v, buf_v, ring_v, sem_a, sem_b, out_sem):
    wid = lax.axis_index("s") * NC + lax.axis_index("c")
    base = wid * B_PER_W
    pltpu.sync_copy(idx_hbm.at[pl.ds(base, B_PER_W)], idx_v)
    pltpu.sync_copy(ord_hbm.at[pl.ds(base, B_PER_W)], ord_v)

    lane_ids = jnp.arange(16, dtype=jnp.int32)
    row_ids = [lane_ids + 16 * t for t in range(EMBED_DIM // 16)]
    zeros16 = jnp.zeros((16,), jnp.int32)

    def at(ref, i):
        return plsc.load_gather(ref, [jnp.full((16,), i, jnp.int32)])[0]

    kf = at(idx_v, 0) >> 7
    klast = at(idx_v, B_PER_W - 1) >> 7
    n_c = lax.div(klast - kf, CHUNK_K) + 1
    kf128 = kf * LANES

    def kstart(c):
        return pl.multiple_of(jnp.minimum(kf + CHUNK_K * c, MAX_K0) * LANES, LANES)

    def fire(c, p, sem):
        pltpu.async_copy(
            table_hbm.at[:, pl.ds(kstart(c), CW)], buf_v.at[p], sem
        )

    def drain(p, sem):
        pltpu.make_async_copy(
            table_hbm.at[:, pl.ds(0, CW)], buf_v.at[p], sem
        ).wait()

    def extract_window(c, p, ptr):
        ks0 = kstart(c)
        be = kf128 + (c + 1) * CW

        def cond(i):
            lab = at(idx_v, jnp.minimum(i, B_PER_W - 1))
            return (i < B_PER_W) & (lab < be)

        def lab_body(i):
            lab = at(idx_v, i)
            bo = at(ord_v, i)
            col = jnp.full((16,), lab - ks0, jnp.int32)
            slot = lax.rem(i, 16)
            sl16 = jnp.full((16,), slot, jnp.int32)
            for t in range(EMBED_DIM // 16):
                vals = plsc.load_gather(buf_v.at[p], [row_ids[t], col])
                plsc.store_scatter(ring_v, [sl16, row_ids[t]], vals)

            @pl.when(i >= 16)
            def _():
                pltpu.make_async_copy(
                    out_hbm.at[pl.ds(base, 1)], ring_v.at[pl.ds(0, 1)], out_sem
                ).wait()

            pltpu.async_copy(
                ring_v.at[pl.ds(slot, 1)], out_hbm.at[pl.ds(bo, 1)], out_sem
            )
            return i + 1

        return lax.while_loop(cond, lab_body, ptr)

    fire(0, 0, sem_a)

    def body(c, ptr):
        def even(ptr):
            @pl.when(c + 1 < n_c)
            def _():
                fire(c + 1, 1, sem_b)

            drain(0, sem_a)
            return extract_window(c, 0, ptr)

        def odd(ptr):
            @pl.when(c + 1 < n_c)
            def _():
                fire(c + 1, 0, sem_a)

            drain(1, sem_b)
            return extract_window(c, 1, ptr)

        return lax.cond(lax.rem(c, 2) == 0, even, odd, ptr)

    lax.fori_loop(0, n_c, body, 0)

    for _ in range(16):
        pltpu.make_async_copy(
            out_hbm.at[pl.ds(base, 1)], ring_v.at[pl.ds(0, 1)], out_sem
        ).wait()


def kernel(labels, embedding_table):
    iota = jnp.arange(BATCH, dtype=jnp.int32)
    slab, order = lax.sort((labels.astype(jnp.int32), iota), num_keys=1)
    return _gather_scan(embedding_table.T, slab, order)
